# baseline (speedup 1.0000x reference)
import functools

import jax
import jax.numpy as jnp
import numpy as np
from jax import lax
from jax.experimental import pallas as pl
from jax.experimental.pallas import tpu as pltpu
from jax.experimental.pallas import tpu_sc as plsc

B, L, V, F = 16384, 20, 100000, 128
NC, NS = 2, 16
NW = NC * NS
ROWS_PER_W = B // NW
IDX_PER_W = ROWS_PER_W * L
SLAB = 128
NSLAB = IDX_PER_W // SLAB

NPHASE = ROWS_PER_W // SLAB
PH_SLABS = NSLAB // NPHASE
_SEG_NP = (
    np.arange(NS, dtype=np.int32)[:, None, None] * (2 * SLAB)
    + np.arange(2, dtype=np.int32)[None, :, None] * SLAB
    + (np.arange(PH_SLABS * SLAB, dtype=np.int32) // L)[None, None, :]
).reshape(NS, 2, PH_SLABS, SLAB)

_EPS = 1e-3
_VEC = 16


_NBUF = 4


def _sc_pool_body(x_r, table, seg, zeros, out, idx_v, seg_v, rows, acc_sh,
                  *sems):
    gsems = sems[:_NBUF]
    dsem = sems[_NBUF:_NBUF + 2]
    zsem = sems[_NBUF + 2:_NBUF + 4]
    cid = lax.axis_index("c")
    sid = lax.axis_index("s")
    wid = sid * NC + cid
    base = sid * (2 * SLAB)

    pltpu.sync_copy(x_r.at[wid], idx_v)
    pltpu.sync_copy(seg.at[sid], seg_v)

    def region(r):
        return acc_sh.at[pl.ds(base + r * SLAB, SLAB)]

    def zero_start(r):
        pltpu.async_copy(zeros, region(r), zsem[r])

    def zero_wait(r):
        pltpu.make_async_copy(zeros, region(r), zsem[r]).wait()

    def drain_start(p, r):
        pltpu.async_copy(region(r),
                         out.at[pl.ds(wid * ROWS_PER_W + p * SLAB, SLAB)],
                         dsem[r])

    def drain_wait(r):
        pltpu.make_async_copy(zeros, region(r), dsem[r]).wait()

    def gather_start(g, b):
        pltpu.async_copy(table.at[idx_v.at[g]], rows.at[b], gsems[b])

    def gather_wait(b):
        pltpu.make_async_copy(table.at[pl.ds(0, SLAB)], rows.at[b],
                              gsems[b]).wait()

    def slab_loop(p, r, lo, hi):
        @pl.loop(lo, hi, step=_NBUF)
        def _slabs(j):
            for b in range(_NBUF):
                gather_wait(b)
                pltpu.sync_copy(rows.at[b], acc_sh.at[seg_v.at[r, j + b]],
                                add=True)
                gather_start(p * PH_SLABS + j + _NBUF + b, b)

    zero_start(0)
    zero_start(1)
    for b in range(_NBUF):
        gather_start(b, b)
    zero_wait(0)

    _MID = 8
    for p in range(NPHASE):
        r = p % 2
        if p > 0:
            zero_wait(r)

        slab_loop(p, r, 0, _MID)
        if 1 <= p <= 2:
            drain_start(p - 1, 1 - r)
            drain_wait(1 - r)
            zero_start(1 - r)
        elif p == NPHASE - 1:
            drain_start(p - 1, 1 - r)
        slab_loop(p, r, _MID,
                  PH_SLABS - _NBUF if p == NPHASE - 1 else PH_SLABS)

        if p == NPHASE - 1:
            for b in range(_NBUF):
                gather_wait(b)
                pltpu.sync_copy(
                    rows.at[b],
                    acc_sh.at[seg_v.at[r, PH_SLABS - _NBUF + b]], add=True)

    plsc.subcore_barrier()
    drain_start(NPHASE - 1, 1)
    drain_wait(0)
    drain_wait(1)


_sc_pool = functools.partial(
    pl.kernel,
    out_type=jax.ShapeDtypeStruct((B, F), jnp.float32),
    mesh=plsc.VectorSubcoreMesh(core_axis_name="c", subcore_axis_name="s",
                                num_cores=NC, num_subcores=NS),
    scratch_types=[
        pltpu.VMEM((NSLAB, SLAB), jnp.int32),
        pltpu.VMEM((2, PH_SLABS, SLAB), jnp.int32),
        pltpu.VMEM((_NBUF, SLAB, F), jnp.float32),
        pltpu.VMEM_SHARED((NS * 2 * SLAB, F), jnp.float32),
    ] + [pltpu.SemaphoreType.DMA] * (_NBUF + 4),
)(_sc_pool_body)


_TC_BLK = 2048
_TC_GRID = B // _TC_BLK


def _tc_body(pooled_hbm, W_ref, prm_ref, out_ref, buf0, buf1, sem0, sem1):
    i = pl.program_id(0)
    bufs = (buf0, buf1)
    sems = (sem0, sem1)

    def fetch(blk, buf, sem):
        pltpu.make_async_copy(
            pooled_hbm.at[pl.ds(blk * _TC_BLK, _TC_BLK)], buf, sem).start()

    @pl.when(i == 0)
    def _():
        fetch(0, buf0, sem0)

    for par in range(2):
        @pl.when(jnp.logical_and(i % 2 == par, i + 1 < _TC_GRID))
        def _():
            nxt = (par + 1) % 2
            fetch(i + 1, bufs[nxt], sems[nxt])

    for par in range(2):
        @pl.when(i % 2 == par)
        def _():
            pltpu.make_async_copy(
                pooled_hbm.at[pl.ds(i * _TC_BLK, _TC_BLK)],
                bufs[par], sems[par]).wait()
            prm = prm_ref[...]
            bias, bng, bnb, bnm, bnv, lng, lnb = (
                prm[k:k + 1] for k in range(7))
            pooled = bufs[par][...] * (1.0 / L)
            z = jnp.dot(pooled, W_ref[...],
                        preferred_element_type=jnp.float32)
            z = z + bias
            z = (z - bnm) / jnp.sqrt(bnv + _EPS) * bng + bnb
            mu = jnp.mean(z, axis=-1, keepdims=True)
            var = jnp.mean(jnp.square(z - mu), axis=-1, keepdims=True)
            out_ref[...] = (z - mu) / jnp.sqrt(var + _EPS) * lng + lnb


def _tc_head(pooled, W, b, bn_gamma, bn_beta, bn_mean, bn_var, ln_gamma, ln_beta):
    prm = jnp.stack([b, bn_gamma, bn_beta, bn_mean, bn_var, ln_gamma,
                     ln_beta, b])
    return pl.pallas_call(
        _tc_body,
        grid=(_TC_GRID,),
        in_specs=[
            pl.BlockSpec(memory_space=pltpu.MemorySpace.HBM),
            pl.BlockSpec((F, F), lambda i: (0, 0)),
            pl.BlockSpec((8, F), lambda i: (0, 0)),
        ],
        out_specs=pl.BlockSpec((_TC_BLK, F), lambda i: (i, 0)),
        out_shape=jax.ShapeDtypeStruct((B, F), jnp.float32),
        scratch_shapes=[
            pltpu.VMEM((_TC_BLK, F), jnp.float32),
            pltpu.VMEM((_TC_BLK, F), jnp.float32),
            pltpu.SemaphoreType.DMA,
            pltpu.SemaphoreType.DMA,
        ],
    )(pooled, W, prm)


def kernel(x, table, W, b, bn_gamma, bn_beta, bn_mean, bn_var, ln_gamma, ln_beta):
    x_r = x.astype(jnp.int32).reshape(NW, NSLAB, SLAB)
    seg = jnp.asarray(_SEG_NP)
    zeros = jnp.zeros((SLAB, F), jnp.float32)
    pooled = _sc_pool(x_r, table, seg, zeros)
    return _tc_head(pooled, W, b, bn_gamma, bn_beta, bn_mean, bn_var,
                    ln_gamma, ln_beta)


# speedup vs baseline: 1.0105x; 1.0105x over previous
import functools

import jax
import jax.numpy as jnp
import numpy as np
from jax import lax
from jax.experimental import pallas as pl
from jax.experimental.pallas import tpu as pltpu
from jax.experimental.pallas import tpu_sc as plsc

B, L, V, F = 16384, 20, 100000, 128
NC, NS = 2, 16
NW = NC * NS
ROWS_PER_W = B // NW
IDX_PER_W = ROWS_PER_W * L
SLAB = 128
NSLAB = IDX_PER_W // SLAB

NPHASE = ROWS_PER_W // SLAB
PH_SLABS = NSLAB // NPHASE
_SEG_NP = (
    np.arange(NS, dtype=np.int32)[:, None, None] * (2 * SLAB)
    + np.arange(2, dtype=np.int32)[None, :, None] * SLAB
    + (np.arange(PH_SLABS * SLAB, dtype=np.int32) // L)[None, None, :]
).reshape(NS, 2, PH_SLABS, SLAB)

_EPS = 1e-3
_VEC = 16


_NBUF = 4


def _sc_pool_body(x_r, table, seg, zeros, out, idx_v, seg_v, rows, acc_sh,
                  *sems):
    gsems = sems[:_NBUF]
    dsem = sems[_NBUF:_NBUF + 2]
    zsem = sems[_NBUF + 2:_NBUF + 4]
    cid = lax.axis_index("c")
    sid = lax.axis_index("s")
    wid = sid * NC + cid
    base = sid * (2 * SLAB)

    pltpu.sync_copy(x_r.at[wid], idx_v)
    pltpu.sync_copy(seg.at[sid], seg_v)

    def region(r):
        return acc_sh.at[pl.ds(base + r * SLAB, SLAB)]

    def zero_start(r):
        pltpu.async_copy(zeros, region(r), zsem[r])

    def zero_wait(r):
        pltpu.make_async_copy(zeros, region(r), zsem[r]).wait()

    def drain_start(p, r):
        pltpu.async_copy(region(r),
                         out.at[pl.ds(wid * ROWS_PER_W + p * SLAB, SLAB)],
                         dsem[r])

    def drain_wait(r):
        pltpu.make_async_copy(zeros, region(r), dsem[r]).wait()

    def gather_start(g, b):
        pltpu.async_copy(table.at[idx_v.at[g]], rows.at[b], gsems[b])

    def gather_wait(b):
        pltpu.make_async_copy(table.at[pl.ds(0, SLAB)], rows.at[b],
                              gsems[b]).wait()

    def slab_loop(p, r, lo, hi):
        @pl.loop(lo, hi, step=_NBUF)
        def _slabs(j):
            for b in range(_NBUF):
                gather_wait(b)
                pltpu.sync_copy(rows.at[b], acc_sh.at[seg_v.at[r, j + b]],
                                add=True)
                gather_start(p * PH_SLABS + j + _NBUF + b, b)

    zero_start(0)
    zero_start(1)
    for b in range(_NBUF):
        gather_start(b, b)
    zero_wait(0)

    _MID = 8
    for p in range(NPHASE):
        r = p % 2
        if p > 0:
            zero_wait(r)

        slab_loop(p, r, 0, _MID)
        if 1 <= p <= 2:
            drain_start(p - 1, 1 - r)
            drain_wait(1 - r)
            zero_start(1 - r)
        elif p == NPHASE - 1:
            drain_start(p - 1, 1 - r)
        slab_loop(p, r, _MID,
                  PH_SLABS - _NBUF if p == NPHASE - 1 else PH_SLABS)

        if p == NPHASE - 1:
            for b in range(_NBUF):
                gather_wait(b)
                pltpu.sync_copy(
                    rows.at[b],
                    acc_sh.at[seg_v.at[r, PH_SLABS - _NBUF + b]], add=True)

    plsc.subcore_barrier()
    drain_start(NPHASE - 1, 1)
    drain_wait(0)
    drain_wait(1)


_sc_pool = functools.partial(
    pl.kernel,
    out_type=jax.ShapeDtypeStruct((B, F), jnp.float32),
    mesh=plsc.VectorSubcoreMesh(core_axis_name="c", subcore_axis_name="s",
                                num_cores=NC, num_subcores=NS),
    scratch_types=[
        pltpu.VMEM((NSLAB, SLAB), jnp.int32),
        pltpu.VMEM((2, PH_SLABS, SLAB), jnp.int32),
        pltpu.VMEM((_NBUF, SLAB, F), jnp.float32),
        pltpu.VMEM_SHARED((NS * 2 * SLAB, F), jnp.float32),
    ] + [pltpu.SemaphoreType.DMA] * (_NBUF + 4),
)(_sc_pool_body)


_TC_BLK = 2048


def _tc_body(pooled_ref, W_ref, prm_ref, out_ref):
    prm = prm_ref[...]
    bias, bng, bnb, bnm, bnv, lng, lnb = (prm[i:i + 1] for i in range(7))
    pooled = pooled_ref[...] * (1.0 / L)
    z = jnp.dot(pooled, W_ref[...], preferred_element_type=jnp.float32)
    z = z + bias
    z = (z - bnm) / jnp.sqrt(bnv + _EPS) * bng + bnb
    mu = jnp.mean(z, axis=-1, keepdims=True)
    var = jnp.mean(jnp.square(z - mu), axis=-1, keepdims=True)
    out_ref[...] = (z - mu) / jnp.sqrt(var + _EPS) * lng + lnb


def _tc_head(pooled, W, b, bn_gamma, bn_beta, bn_mean, bn_var, ln_gamma, ln_beta):
    prm = jnp.stack([b, bn_gamma, bn_beta, bn_mean, bn_var, ln_gamma,
                     ln_beta, b])
    return pl.pallas_call(
        _tc_body,
        grid=(B // _TC_BLK,),
        in_specs=[
            pl.BlockSpec((_TC_BLK, F), lambda i: (i, 0)),
            pl.BlockSpec((F, F), lambda i: (0, 0)),
            pl.BlockSpec((8, F), lambda i: (0, 0)),
        ],
        out_specs=pl.BlockSpec((_TC_BLK, F), lambda i: (i, 0)),
        out_shape=jax.ShapeDtypeStruct((B, F), jnp.float32),
        input_output_aliases={0: 0},
    )(pooled, W, prm)


def kernel(x, table, W, b, bn_gamma, bn_beta, bn_mean, bn_var, ln_gamma, ln_beta):
    x_r = x.astype(jnp.int32).reshape(NW, NSLAB, SLAB)
    seg = jnp.asarray(_SEG_NP)
    zeros = jnp.zeros((SLAB, F), jnp.float32)
    pooled = _sc_pool(x_r, table, seg, zeros)
    return _tc_head(pooled, W, b, bn_gamma, bn_beta, bn_mean, bn_var,
                    ln_gamma, ln_beta)
